# Initial kernel scaffold; baseline (speedup 1.0000x reference)
import jax
import jax.numpy as jnp
from jax.experimental import pallas as pl


def kernel(node_feat, edge_index, edge_feat, x_noisy, t01, node_w, node_b, color_w, color_b, time_w, time_b, msg_w1, msg_b1, msg_w2, msg_b2, upd_w1, upd_b1, upd_w2, upd_b2, out_w1, out_b1, out_w2, out_b2):
    raise NotImplementedError("write your pallas kernel here")


# speedup vs baseline: 2.2707x; 2.2707x over previous
import functools
import math

import jax
import jax.numpy as jnp
from jax import lax
from jax.experimental import pallas as pl
from jax.experimental.pallas import tpu as pltpu
from jax.experimental.pallas import tpu_sc as plsc

N = 10000
E = 320000
NODE_IN = 128
EDGE_IN = 16
TIME_DIM = 32
HDIM = 64
LAYERS = 4

_NC = 2
_NS = 16
_NW = _NC * _NS

_IW = 100
_GSUB = 8
_GC = _GSUB * _IW
_GW = (2 * E) // _NW
_SSUB = 2
_SC = _SSUB * _IW
_SW = E // _NS

_NZ = 624
_NZT = N - _NS * _NZ


def _silu(x):
    return x / (1.0 + jnp.exp(-x))


def _embed_body(nf_ref, xn_ref, t_ref, nw_ref, cw_ref, tw_ref, fr_ref,
                b_ref, wab_ref, h_ref, p_ref):
    ang = t_ref[...] * fr_ref[...]
    temb = jnp.concatenate([jnp.sin(ang), jnp.cos(ang)], axis=1)
    h = (jnp.dot(nf_ref[...], nw_ref[...], preferred_element_type=jnp.float32)
         + jnp.dot(xn_ref[...], cw_ref[...], preferred_element_type=jnp.float32)
         + jnp.dot(temb, tw_ref[...], preferred_element_type=jnp.float32)
         + b_ref[...])
    h_ref[...] = h
    p_ref[...] = jnp.dot(h, wab_ref[...], preferred_element_type=jnp.float32)


def _edge_body(g_ref, ef_ref, wc_ref, b1_ref, w2_ref, b2_ref, m_ref):
    g = g_ref[...]
    ec = jnp.dot(ef_ref[...], wc_ref[...],
                 preferred_element_type=jnp.float32) + b1_ref[...]
    z1 = _silu(g[:, 0:HDIM] + g[:, 3 * HDIM:4 * HDIM] + ec)
    z2 = _silu(g[:, 2 * HDIM:3 * HDIM] + g[:, HDIM:2 * HDIM] + ec)
    w2 = w2_ref[...]
    b2 = b2_ref[...]
    mij = _silu(jnp.dot(z1, w2, preferred_element_type=jnp.float32) + b2)
    mji = _silu(jnp.dot(z2, w2, preferred_element_type=jnp.float32) + b2)
    m_ref[...] = jnp.concatenate([mij, mji], axis=1)


def _update_body(h_ref, agg2_ref, w1h_ref, w1a_ref, b1_ref, w2_ref, b2_ref,
                 wab_ref, out_ref, p_ref):
    h = h_ref[...]
    a2 = agg2_ref[...]
    agg = a2[:N, :HDIM] + a2[N:, HDIM:]
    u = _silu(jnp.dot(h, w1h_ref[...], preferred_element_type=jnp.float32)
              + jnp.dot(agg, w1a_ref[...], preferred_element_type=jnp.float32)
              + b1_ref[...])
    u = _silu(jnp.dot(u, w2_ref[...], preferred_element_type=jnp.float32)
              + b2_ref[...])
    hn = h + u
    out_ref[...] = hn
    p_ref[...] = jnp.dot(hn, wab_ref[...], preferred_element_type=jnp.float32)


def _head_body(h_ref, w1_ref, b1_ref, w2_ref, b2_ref, out_ref):
    z = _silu(jnp.dot(h_ref[...], w1_ref[...],
                      preferred_element_type=jnp.float32) + b1_ref[...])
    out_ref[...] = jnp.dot(z, w2_ref[...],
                           preferred_element_type=jnp.float32) + b2_ref[...]


_MESH = plsc.VectorSubcoreMesh(core_axis_name="c", subcore_axis_name="s")


@functools.partial(
    pl.kernel,
    out_type=jax.ShapeDtypeStruct((2 * E, 2 * HDIM), jnp.float32),
    mesh=_MESH,
    scratch_types=[
        pltpu.VMEM((_GSUB, 1, _IW), jnp.int32),
        pltpu.VMEM((_GC, 2 * HDIM), jnp.float32),
        pltpu.SemaphoreType.DMA,
    ],
)
def _gather_call(p_hbm, idx_hbm, out_hbm, idx_v, rows_v, sem):
    c = lax.axis_index("c")
    s = lax.axis_index("s")
    w = c * _NS + s
    irow0 = w * (_GW // _IW)
    row0 = w * _GW

    def chunk(t, carry):
        pltpu.sync_copy(idx_hbm.at[pl.ds(irow0 + t * _GSUB, _GSUB)], idx_v)
        cps = [
            pltpu.async_copy(p_hbm.at[idx_v.at[sb, 0]],
                             rows_v.at[pl.ds(sb * _IW, _IW)], sem)
            for sb in range(_GSUB)
        ]
        for cp in cps:
            cp.wait()
        pltpu.sync_copy(rows_v, out_hbm.at[pl.ds(row0 + t * _GC, _GC)])
        return carry

    lax.fori_loop(0, _GW // _GC, chunk, 0)


@functools.partial(
    pl.kernel,
    out_type=jax.ShapeDtypeStruct((2 * N, 2 * HDIM), jnp.float32),
    mesh=_MESH,
    scratch_types=[
        pltpu.VMEM((_SSUB, 1, _IW), jnp.int32),
        pltpu.VMEM((_SC, 2 * HDIM), jnp.float32),
        pltpu.VMEM_SHARED((N, 2 * HDIM), jnp.float32),
    ],
)
def _scatter_call(m_hbm, iidx_hbm, jidx_hbm, zeros_hbm, out_hbm,
                  idx_v, vals_v, agg_sh):
    c = lax.axis_index("c")
    s = lax.axis_index("s")
    pltpu.sync_copy(zeros_hbm, agg_sh.at[pl.ds(s * _NZ, _NZ)])

    @pl.when(s == 0)
    def _():
        pltpu.sync_copy(zeros_hbm.at[pl.ds(0, _NZT)],
                        agg_sh.at[pl.ds(_NS * _NZ, _NZT)])

    plsc.subcore_barrier()

    def do(idx_hbm):
        irow0 = s * (_SW // _IW)
        row0 = s * _SW

        def chunk(t, carry):
            pltpu.sync_copy(idx_hbm.at[pl.ds(irow0 + t * _SSUB, _SSUB)], idx_v)
            pltpu.sync_copy(m_hbm.at[pl.ds(row0 + t * _SC, _SC)], vals_v)
            for sb in range(_SSUB):
                pltpu.sync_copy(vals_v.at[pl.ds(sb * _IW, _IW)],
                                agg_sh.at[idx_v.at[sb, 0]], add=True)
            return carry

        lax.fori_loop(0, _SW // _SC, chunk, 0)

    @pl.when(c == 0)
    def _():
        do(iidx_hbm)

    @pl.when(c == 1)
    def _():
        do(jidx_hbm)

    plsc.subcore_barrier()
    pltpu.sync_copy(agg_sh.at[pl.ds(s * _NZ, _NZ)],
                    out_hbm.at[pl.ds(c * N + s * _NZ, _NZ)])

    @pl.when(s == 0)
    def _():
        pltpu.sync_copy(agg_sh.at[pl.ds(_NS * _NZ, _NZT)],
                        out_hbm.at[pl.ds(c * N + _NS * _NZ, _NZT)])


def _embed_call(node_feat, x_noisy, t2d, node_w, color_w, time_w, freqs,
                bias, wab):
    return pl.pallas_call(
        _embed_body,
        out_shape=[
            jax.ShapeDtypeStruct((N, HDIM), jnp.float32),
            jax.ShapeDtypeStruct((N, 2 * HDIM), jnp.float32),
        ],
    )(node_feat, x_noisy, t2d, node_w, color_w, time_w, freqs, bias, wab)


_EB = 5000


def _edge_call(g, edge_feat, wc, b1, w2, b2):
    nblk = E // _EB
    return pl.pallas_call(
        _edge_body,
        grid=(nblk,),
        in_specs=[
            pl.BlockSpec((_EB, 4 * HDIM), lambda b: (b, 0)),
            pl.BlockSpec((_EB, EDGE_IN), lambda b: (b, 0)),
            pl.BlockSpec((EDGE_IN, HDIM), lambda b: (0, 0)),
            pl.BlockSpec((1, HDIM), lambda b: (0, 0)),
            pl.BlockSpec((HDIM, HDIM), lambda b: (0, 0)),
            pl.BlockSpec((1, HDIM), lambda b: (0, 0)),
        ],
        out_specs=pl.BlockSpec((_EB, 2 * HDIM), lambda b: (b, 0)),
        out_shape=jax.ShapeDtypeStruct((E, 2 * HDIM), jnp.float32),
    )(g, edge_feat, wc, b1, w2, b2)


def _update_call(h, agg2, w1h, w1a, b1, w2, b2, wab):
    return pl.pallas_call(
        _update_body,
        out_shape=[
            jax.ShapeDtypeStruct((N, HDIM), jnp.float32),
            jax.ShapeDtypeStruct((N, 2 * HDIM), jnp.float32),
        ],
    )(h, agg2, w1h, w1a, b1, w2, b2, wab)


def _head_call(h, w1, b1, w2, b2):
    return pl.pallas_call(
        _head_body,
        out_shape=jax.ShapeDtypeStruct((N, 3), jnp.float32),
    )(h, w1, b1, w2, b2)


def kernel(node_feat, edge_index, edge_feat, x_noisy, t01, node_w, node_b,
           color_w, color_b, time_w, time_b, msg_w1, msg_b1, msg_w2, msg_b2,
           upd_w1, upd_b1, upd_w2, upd_b2, out_w1, out_b1, out_w2, out_b2):
    half = TIME_DIM // 2
    freqs = jnp.exp(jnp.linspace(math.log(1.0), math.log(1000.0), half))
    freqs2d = (freqs * (2.0 * math.pi)).reshape(1, half).astype(jnp.float32)
    t2d = t01.reshape(N, 1)
    bias0 = (node_b + color_b + time_b).reshape(1, HDIM)

    wab = jnp.concatenate([msg_w1[:, :HDIM], msg_w1[:, HDIM:2 * HDIM]],
                          axis=2)

    fe3d = edge_index.reshape((2 * E) // _IW, 1, _IW)
    iidx3d = edge_index[:, 0].reshape(E // _IW, 1, _IW)
    jidx3d = edge_index[:, 1].reshape(E // _IW, 1, _IW)
    zeros = jnp.zeros((_NZ, 2 * HDIM), jnp.float32)

    h, p = _embed_call(node_feat, x_noisy, t2d, node_w, color_w, time_w,
                       freqs2d, bias0, wab[0])

    for k in range(LAYERS):
        g2 = _gather_call(p, fe3d)
        g = g2.reshape(E, 4 * HDIM)
        m = _edge_call(g, edge_feat, msg_w1[k, 2 * HDIM:],
                       msg_b1[k].reshape(1, HDIM), msg_w2[k],
                       msg_b2[k].reshape(1, HDIM))
        agg2 = _scatter_call(m, iidx3d, jidx3d, zeros)
        wab_next = wab[min(k + 1, LAYERS - 1)]
        h, p = _update_call(h, agg2, upd_w1[k, :HDIM], upd_w1[k, HDIM:],
                            upd_b1[k].reshape(1, HDIM), upd_w2[k],
                            upd_b2[k].reshape(1, HDIM), wab_next)

    return _head_call(h, out_w1, out_b1.reshape(1, HDIM), out_w2,
                      out_b2.reshape(1, 3))


# speedup vs baseline: 2.5344x; 1.1161x over previous
import functools
import math

import jax
import jax.numpy as jnp
from jax import lax
from jax.experimental import pallas as pl
from jax.experimental.pallas import tpu as pltpu
from jax.experimental.pallas import tpu_sc as plsc

N = 10000
E = 320000
NODE_IN = 128
EDGE_IN = 16
TIME_DIM = 32
HDIM = 64
LAYERS = 4

_NC = 2
_NS = 16
_NW = _NC * _NS

_IW = 100
_GSUB = 4
_GC = _GSUB * _IW
_GW = (2 * E) // _NW
_GNCH = _GW // _GC
_SIW = 80
_SSUB = 2
_SC = _SSUB * _SIW
_SW = E // _NS
_SNCH = _SW // _SC

_NZ = 624
_NZT = N - _NS * _NZ


def _silu(x):
    return x / (1.0 + jnp.exp(-x))


def _embed_body(nf_ref, xn_ref, t_ref, nw_ref, cw_ref, tw_ref, fr_ref,
                b_ref, wab_ref, h_ref, p_ref):
    ang = t_ref[...] * fr_ref[...]
    temb = jnp.concatenate([jnp.sin(ang), jnp.cos(ang)], axis=1)
    h = (jnp.dot(nf_ref[...], nw_ref[...], preferred_element_type=jnp.float32)
         + jnp.dot(xn_ref[...], cw_ref[...], preferred_element_type=jnp.float32)
         + jnp.dot(temb, tw_ref[...], preferred_element_type=jnp.float32)
         + b_ref[...])
    h_ref[...] = h
    p_ref[...] = jnp.dot(h, wab_ref[...], preferred_element_type=jnp.float32)


def _edge_body(g_ref, ef_ref, wc_ref, b1_ref, w2_ref, b2_ref, m_ref):
    g = g_ref[...]
    ec = jnp.dot(ef_ref[...], wc_ref[...],
                 preferred_element_type=jnp.float32) + b1_ref[...]
    z1 = _silu(g[:, 0:HDIM] + g[:, 3 * HDIM:4 * HDIM] + ec)
    z2 = _silu(g[:, 2 * HDIM:3 * HDIM] + g[:, HDIM:2 * HDIM] + ec)
    w2 = w2_ref[...]
    b2 = b2_ref[...]
    mij = _silu(jnp.dot(z1, w2, preferred_element_type=jnp.float32) + b2)
    mji = _silu(jnp.dot(z2, w2, preferred_element_type=jnp.float32) + b2)
    m_ref[...] = jnp.concatenate([mij, mji], axis=1)


def _update_body(h_ref, agg2_ref, w1h_ref, w1a_ref, b1_ref, w2_ref, b2_ref,
                 wab_ref, out_ref, p_ref):
    h = h_ref[...]
    a2 = agg2_ref[...]
    agg = a2[:N, :HDIM] + a2[N:, HDIM:]
    u = _silu(jnp.dot(h, w1h_ref[...], preferred_element_type=jnp.float32)
              + jnp.dot(agg, w1a_ref[...], preferred_element_type=jnp.float32)
              + b1_ref[...])
    u = _silu(jnp.dot(u, w2_ref[...], preferred_element_type=jnp.float32)
              + b2_ref[...])
    hn = h + u
    out_ref[...] = hn
    p_ref[...] = jnp.dot(hn, wab_ref[...], preferred_element_type=jnp.float32)


def _head_body(h_ref, w1_ref, b1_ref, w2_ref, b2_ref, out_ref):
    z = _silu(jnp.dot(h_ref[...], w1_ref[...],
                      preferred_element_type=jnp.float32) + b1_ref[...])
    out_ref[...] = jnp.dot(z, w2_ref[...],
                           preferred_element_type=jnp.float32) + b2_ref[...]


_MESH = plsc.VectorSubcoreMesh(core_axis_name="c", subcore_axis_name="s")


@functools.partial(
    pl.kernel,
    out_type=jax.ShapeDtypeStruct((2 * E, 2 * HDIM), jnp.float32),
    mesh=_MESH,
    scratch_types=[
        pltpu.VMEM((_GSUB, 1, _IW), jnp.int32),
        pltpu.VMEM((_GSUB, 1, _IW), jnp.int32),
        pltpu.VMEM((_GC, 2 * HDIM), jnp.float32),
        pltpu.VMEM((_GC, 2 * HDIM), jnp.float32),
        pltpu.SemaphoreType.DMA,
        pltpu.SemaphoreType.DMA,
        pltpu.SemaphoreType.DMA,
    ],
)
def _gather_call(p_hbm, idx_hbm, out_hbm, idx_v0, idx_v1, rows_v0, rows_v1,
                 gsem, wsem0, wsem1):
    c = lax.axis_index("c")
    s = lax.axis_index("s")
    w = c * _NS + s
    irow0 = w * (_GW // _IW)
    row0 = w * _GW
    bufs = ((idx_v0, rows_v0, wsem0), (idx_v1, rows_v1, wsem1))

    def chunk(u, carry):
        for b in range(2):
            idx_v, rows_v, wsem = bufs[b]
            t = 2 * u + b

            @pl.when(t >= 2)
            def _():
                pltpu.make_async_copy(
                    rows_v, out_hbm.at[pl.ds(row0, _GC)], wsem).wait()

            pltpu.sync_copy(idx_hbm.at[pl.ds(irow0 + t * _GSUB, _GSUB)],
                            idx_v)
            cps = [
                pltpu.async_copy(p_hbm.at[idx_v.at[sb, 0]],
                                 rows_v.at[pl.ds(sb * _IW, _IW)], gsem)
                for sb in range(_GSUB)
            ]
            for cp in cps:
                cp.wait()
            pltpu.async_copy(rows_v, out_hbm.at[pl.ds(row0 + t * _GC, _GC)],
                             wsem)
        return carry

    lax.fori_loop(0, _GNCH // 2, chunk, 0)
    for b in range(2):
        idx_v, rows_v, wsem = bufs[b]
        pltpu.make_async_copy(rows_v, out_hbm.at[pl.ds(row0, _GC)],
                              wsem).wait()


@functools.partial(
    pl.kernel,
    out_type=jax.ShapeDtypeStruct((2 * N, 2 * HDIM), jnp.float32),
    mesh=_MESH,
    scratch_types=[
        pltpu.VMEM((_SSUB, 1, _SIW), jnp.int32),
        pltpu.VMEM((_SSUB, 1, _SIW), jnp.int32),
        pltpu.VMEM((_SC, 2 * HDIM), jnp.float32),
        pltpu.VMEM((_SC, 2 * HDIM), jnp.float32),
        pltpu.SemaphoreType.DMA,
        pltpu.SemaphoreType.DMA,
        pltpu.SemaphoreType.DMA,
        pltpu.SemaphoreType.DMA,
        pltpu.VMEM_SHARED((N, 2 * HDIM), jnp.float32),
    ],
)
def _scatter_call(m_hbm, iidx_hbm, jidx_hbm, zeros_hbm, out_hbm,
                  idx_v0, idx_v1, vals_v0, vals_v1, lsem0, lsem1,
                  ssem0, ssem1, agg_sh):
    c = lax.axis_index("c")
    s = lax.axis_index("s")
    pltpu.sync_copy(zeros_hbm, agg_sh.at[pl.ds(s * _NZ, _NZ)])

    @pl.when(s == 0)
    def _():
        pltpu.sync_copy(zeros_hbm.at[pl.ds(0, _NZT)],
                        agg_sh.at[pl.ds(_NS * _NZ, _NZT)])

    plsc.subcore_barrier()

    def do(idx_hbm):
        irow0 = s * (_SW // _SIW)
        row0 = s * _SW
        bufs = ((idx_v0, vals_v0, lsem0, ssem0),
                (idx_v1, vals_v1, lsem1, ssem1))

        def load(t, bi):
            idx_v, vals_v, lsem, _ = bufs[bi]
            pltpu.async_copy(idx_hbm.at[pl.ds(irow0 + t * _SSUB, _SSUB)],
                             idx_v, lsem)
            pltpu.async_copy(m_hbm.at[pl.ds(row0 + t * _SC, _SC)], vals_v,
                             lsem)

        def drain_streams(bi):
            idx_v, vals_v, _, ssem = bufs[bi]
            for sb in range(_SSUB):
                pltpu.make_async_copy(vals_v.at[pl.ds(sb * _SIW, _SIW)],
                                      agg_sh.at[idx_v.at[sb, 0]],
                                      ssem).wait()

        def run(t, b):
            idx_v, vals_v, lsem, ssem = bufs[b]
            pltpu.make_async_copy(
                idx_hbm.at[pl.ds(irow0, _SSUB)], idx_v, lsem).wait()
            pltpu.make_async_copy(
                m_hbm.at[pl.ds(row0, _SC)], vals_v, lsem).wait()
            for sb in range(_SSUB):
                pltpu.async_copy(vals_v.at[pl.ds(sb * _SIW, _SIW)],
                                 agg_sh.at[idx_v.at[sb, 0]], ssem, add=True)

            @pl.when(t >= 1)
            def _():
                drain_streams(1 - b)

            @pl.when(t + 1 < _SNCH)
            def _():
                load(t + 1, 1 - b)

        load(0, 0)

        def chunk(u, carry):
            run(2 * u, 0)
            run(2 * u + 1, 1)
            return carry

        lax.fori_loop(0, _SNCH // 2, chunk, 0)
        run(_SNCH - 1, 0)
        drain_streams(0)

    @pl.when(c == 0)
    def _():
        do(iidx_hbm)

    @pl.when(c == 1)
    def _():
        do(jidx_hbm)

    plsc.subcore_barrier()
    pltpu.sync_copy(agg_sh.at[pl.ds(s * _NZ, _NZ)],
                    out_hbm.at[pl.ds(c * N + s * _NZ, _NZ)])

    @pl.when(s == 0)
    def _():
        pltpu.sync_copy(agg_sh.at[pl.ds(_NS * _NZ, _NZT)],
                        out_hbm.at[pl.ds(c * N + _NS * _NZ, _NZT)])


def _embed_call(node_feat, x_noisy, t2d, node_w, color_w, time_w, freqs,
                bias, wab):
    return pl.pallas_call(
        _embed_body,
        out_shape=[
            jax.ShapeDtypeStruct((N, HDIM), jnp.float32),
            jax.ShapeDtypeStruct((N, 2 * HDIM), jnp.float32),
        ],
    )(node_feat, x_noisy, t2d, node_w, color_w, time_w, freqs, bias, wab)


_EB = 5000


def _edge_call(g, edge_feat, wc, b1, w2, b2):
    nblk = E // _EB
    return pl.pallas_call(
        _edge_body,
        grid=(nblk,),
        in_specs=[
            pl.BlockSpec((_EB, 4 * HDIM), lambda b: (b, 0)),
            pl.BlockSpec((_EB, EDGE_IN), lambda b: (b, 0)),
            pl.BlockSpec((EDGE_IN, HDIM), lambda b: (0, 0)),
            pl.BlockSpec((1, HDIM), lambda b: (0, 0)),
            pl.BlockSpec((HDIM, HDIM), lambda b: (0, 0)),
            pl.BlockSpec((1, HDIM), lambda b: (0, 0)),
        ],
        out_specs=pl.BlockSpec((_EB, 2 * HDIM), lambda b: (b, 0)),
        out_shape=jax.ShapeDtypeStruct((E, 2 * HDIM), jnp.float32),
    )(g, edge_feat, wc, b1, w2, b2)


def _update_call(h, agg2, w1h, w1a, b1, w2, b2, wab):
    return pl.pallas_call(
        _update_body,
        out_shape=[
            jax.ShapeDtypeStruct((N, HDIM), jnp.float32),
            jax.ShapeDtypeStruct((N, 2 * HDIM), jnp.float32),
        ],
    )(h, agg2, w1h, w1a, b1, w2, b2, wab)


def _head_call(h, w1, b1, w2, b2):
    return pl.pallas_call(
        _head_body,
        out_shape=jax.ShapeDtypeStruct((N, 3), jnp.float32),
    )(h, w1, b1, w2, b2)


def kernel(node_feat, edge_index, edge_feat, x_noisy, t01, node_w, node_b,
           color_w, color_b, time_w, time_b, msg_w1, msg_b1, msg_w2, msg_b2,
           upd_w1, upd_b1, upd_w2, upd_b2, out_w1, out_b1, out_w2, out_b2):
    half = TIME_DIM // 2
    freqs = jnp.exp(jnp.linspace(math.log(1.0), math.log(1000.0), half))
    freqs2d = (freqs * (2.0 * math.pi)).reshape(1, half).astype(jnp.float32)
    t2d = t01.reshape(N, 1)
    bias0 = (node_b + color_b + time_b).reshape(1, HDIM)

    wab = jnp.concatenate([msg_w1[:, :HDIM], msg_w1[:, HDIM:2 * HDIM]],
                          axis=2)

    fe3d = edge_index.reshape((2 * E) // _IW, 1, _IW)
    iidx3d = edge_index[:, 0].reshape(E // _SIW, 1, _SIW)
    jidx3d = edge_index[:, 1].reshape(E // _SIW, 1, _SIW)
    zeros = jnp.zeros((_NZ, 2 * HDIM), jnp.float32)

    h, p = _embed_call(node_feat, x_noisy, t2d, node_w, color_w, time_w,
                       freqs2d, bias0, wab[0])

    for k in range(LAYERS):
        g2 = _gather_call(p, fe3d)
        g = g2.reshape(E, 4 * HDIM)
        m = _edge_call(g, edge_feat, msg_w1[k, 2 * HDIM:],
                       msg_b1[k].reshape(1, HDIM), msg_w2[k],
                       msg_b2[k].reshape(1, HDIM))
        agg2 = _scatter_call(m, iidx3d, jidx3d, zeros)
        wab_next = wab[min(k + 1, LAYERS - 1)]
        h, p = _update_call(h, agg2, upd_w1[k, :HDIM], upd_w1[k, HDIM:],
                            upd_b1[k].reshape(1, HDIM), upd_w2[k],
                            upd_b2[k].reshape(1, HDIM), wab_next)

    return _head_call(h, out_w1, out_b1.reshape(1, HDIM), out_w2,
                      out_b2.reshape(1, 3))


# speedup vs baseline: 4.2418x; 1.6737x over previous
import functools
import math

import jax
import jax.numpy as jnp
from jax import lax
from jax.experimental import pallas as pl
from jax.experimental.pallas import tpu as pltpu
from jax.experimental.pallas import tpu_sc as plsc

N = 10000
E = 320000
NODE_IN = 128
EDGE_IN = 16
TIME_DIM = 32
HDIM = 64
LAYERS = 4

_NC = 2
_NS = 16
_NW = _NC * _NS

_IW = 80
_GSUB = 5
_GC = _GSUB * _IW
_GW = E // _NW
_GNCH = _GW // _GC
_SIW = 80
_SSUB = 2
_SC = _SSUB * _SIW
_SW = E // _NS
_SNCH = _SW // _SC

_NZ = 624
_NZT = N - _NS * _NZ


def _silu(x):
    return x / (1.0 + jnp.exp(-x))


def _embed_body(nf_ref, xn_ref, t_ref, nw_ref, cw_ref, tw_ref, fr_ref,
                b_ref, wab_ref, wba_ref, h_ref, p_ref, p2_ref):
    ang = t_ref[...] * fr_ref[...]
    temb = jnp.concatenate([jnp.sin(ang), jnp.cos(ang)], axis=1)
    h = (jnp.dot(nf_ref[...], nw_ref[...], preferred_element_type=jnp.float32)
         + jnp.dot(xn_ref[...], cw_ref[...], preferred_element_type=jnp.float32)
         + jnp.dot(temb, tw_ref[...], preferred_element_type=jnp.float32)
         + b_ref[...])
    h_ref[...] = h
    p_ref[...] = jnp.dot(h, wab_ref[...], preferred_element_type=jnp.float32)
    p2_ref[...] = jnp.dot(h, wba_ref[...], preferred_element_type=jnp.float32)


def _edge_body(g_ref, ef_ref, wc_ref, b1_ref, w2_ref, b2_ref, m_ref):
    g = g_ref[...]
    ec = jnp.dot(ef_ref[...], wc_ref[...],
                 preferred_element_type=jnp.float32) + b1_ref[...]
    z1 = _silu(g[:, :HDIM] + ec)
    z2 = _silu(g[:, HDIM:] + ec)
    w2 = w2_ref[...]
    b2 = b2_ref[...]
    mij = _silu(jnp.dot(z1, w2, preferred_element_type=jnp.float32) + b2)
    mji = _silu(jnp.dot(z2, w2, preferred_element_type=jnp.float32) + b2)
    m_ref[...] = jnp.concatenate([mij, mji], axis=1)


def _update_body(h_ref, agg2_ref, w1h_ref, w1a_ref, b1_ref, w2_ref, b2_ref,
                 wab_ref, wba_ref, out_ref, p_ref, p2_ref):
    h = h_ref[...]
    a2 = agg2_ref[...]
    agg = a2[:N, :HDIM] + a2[N:, HDIM:]
    u = _silu(jnp.dot(h, w1h_ref[...], preferred_element_type=jnp.float32)
              + jnp.dot(agg, w1a_ref[...], preferred_element_type=jnp.float32)
              + b1_ref[...])
    u = _silu(jnp.dot(u, w2_ref[...], preferred_element_type=jnp.float32)
              + b2_ref[...])
    hn = h + u
    out_ref[...] = hn
    p_ref[...] = jnp.dot(hn, wab_ref[...], preferred_element_type=jnp.float32)
    p2_ref[...] = jnp.dot(hn, wba_ref[...],
                          preferred_element_type=jnp.float32)


def _head_body(h_ref, w1_ref, b1_ref, w2_ref, b2_ref, out_ref):
    z = _silu(jnp.dot(h_ref[...], w1_ref[...],
                      preferred_element_type=jnp.float32) + b1_ref[...])
    out_ref[...] = jnp.dot(z, w2_ref[...],
                           preferred_element_type=jnp.float32) + b2_ref[...]


_MESH = plsc.VectorSubcoreMesh(core_axis_name="c", subcore_axis_name="s")


@functools.partial(
    pl.kernel,
    out_type=jax.ShapeDtypeStruct((E, 2 * HDIM), jnp.float32),
    mesh=_MESH,
    scratch_types=[
        pltpu.VMEM((_GSUB, 1, _IW), jnp.int32),
        pltpu.VMEM((_GSUB, 1, _IW), jnp.int32),
        pltpu.VMEM((_GSUB, 1, _IW), jnp.int32),
        pltpu.VMEM((_GSUB, 1, _IW), jnp.int32),
        pltpu.VMEM((_GC, 2 * HDIM), jnp.float32),
        pltpu.VMEM((_GC, 2 * HDIM), jnp.float32),
        pltpu.SemaphoreType.DMA,
        pltpu.SemaphoreType.DMA,
        pltpu.SemaphoreType.DMA,
        pltpu.SemaphoreType.DMA,
        pltpu.SemaphoreType.DMA,
    ],
)
def _gather_call(p_hbm, p2_hbm, iidx_hbm, jidx_hbm, out_hbm,
                 idxi0, idxi1, idxj0, idxj1, rows_v0, rows_v1,
                 gsem, wsem0, wsem1, lsem0, lsem1):
    c = lax.axis_index("c")
    s = lax.axis_index("s")
    w = c * _NS + s
    irow0 = w * (_GW // _IW)
    row0 = w * _GW
    bufs = ((idxi0, idxj0, rows_v0, wsem0, lsem0),
            (idxi1, idxj1, rows_v1, wsem1, lsem1))

    def load_idx(t, bi):
        idxi, idxj, _, _, lsem = bufs[bi]
        pltpu.async_copy(iidx_hbm.at[pl.ds(irow0 + t * _GSUB, _GSUB)], idxi,
                         lsem)
        pltpu.async_copy(jidx_hbm.at[pl.ds(irow0 + t * _GSUB, _GSUB)], idxj,
                         lsem)

    def run(t, b):
        idxi, idxj, rows_v, wsem, lsem = bufs[b]

        @pl.when(t + 1 < _GNCH)
        def _():
            load_idx(t + 1, 1 - b)

        @pl.when(t >= 2)
        def _():
            pltpu.make_async_copy(
                rows_v, out_hbm.at[pl.ds(row0, _GC)], wsem).wait()

        pltpu.make_async_copy(iidx_hbm.at[pl.ds(irow0, _GSUB)], idxi,
                              lsem).wait()
        pltpu.make_async_copy(jidx_hbm.at[pl.ds(irow0, _GSUB)], idxj,
                              lsem).wait()
        base = [
            pltpu.async_copy(p_hbm.at[idxi.at[sb, 0]],
                             rows_v.at[pl.ds(sb * _IW, _IW)], gsem)
            for sb in range(_GSUB)
        ]
        for cp in base:
            cp.wait()
        adds = [
            pltpu.async_copy(p2_hbm.at[idxj.at[sb, 0]],
                             rows_v.at[pl.ds(sb * _IW, _IW)], gsem, add=True)
            for sb in range(_GSUB)
        ]
        for cp in adds:
            cp.wait()
        pltpu.async_copy(rows_v, out_hbm.at[pl.ds(row0 + t * _GC, _GC)],
                         wsem)

    load_idx(0, 0)

    def chunk(u, carry):
        run(2 * u, 0)
        run(2 * u + 1, 1)
        return carry

    lax.fori_loop(0, _GNCH // 2, chunk, 0)
    run(_GNCH - 1, 0)
    for b in range(2):
        idxi, idxj, rows_v, wsem, lsem = bufs[b]
        pltpu.make_async_copy(rows_v, out_hbm.at[pl.ds(row0, _GC)],
                              wsem).wait()


@functools.partial(
    pl.kernel,
    out_type=jax.ShapeDtypeStruct((2 * N, 2 * HDIM), jnp.float32),
    mesh=_MESH,
    scratch_types=[
        pltpu.VMEM((_SSUB, 1, _SIW), jnp.int32),
        pltpu.VMEM((_SSUB, 1, _SIW), jnp.int32),
        pltpu.VMEM((_SC, 2 * HDIM), jnp.float32),
        pltpu.VMEM((_SC, 2 * HDIM), jnp.float32),
        pltpu.SemaphoreType.DMA,
        pltpu.SemaphoreType.DMA,
        pltpu.SemaphoreType.DMA,
        pltpu.SemaphoreType.DMA,
        pltpu.VMEM_SHARED((N, 2 * HDIM), jnp.float32),
    ],
)
def _scatter_call(m_hbm, iidx_hbm, jidx_hbm, zeros_hbm, out_hbm,
                  idx_v0, idx_v1, vals_v0, vals_v1, lsem0, lsem1,
                  ssem0, ssem1, agg_sh):
    c = lax.axis_index("c")
    s = lax.axis_index("s")
    pltpu.sync_copy(zeros_hbm, agg_sh.at[pl.ds(s * _NZ, _NZ)])

    @pl.when(s == 0)
    def _():
        pltpu.sync_copy(zeros_hbm.at[pl.ds(0, _NZT)],
                        agg_sh.at[pl.ds(_NS * _NZ, _NZT)])

    plsc.subcore_barrier()

    def do(idx_hbm):
        irow0 = s * (_SW // _SIW)
        row0 = s * _SW
        bufs = ((idx_v0, vals_v0, lsem0, ssem0),
                (idx_v1, vals_v1, lsem1, ssem1))

        def load(t, bi):
            idx_v, vals_v, lsem, _ = bufs[bi]
            pltpu.async_copy(idx_hbm.at[pl.ds(irow0 + t * _SSUB, _SSUB)],
                             idx_v, lsem)
            pltpu.async_copy(m_hbm.at[pl.ds(row0 + t * _SC, _SC)], vals_v,
                             lsem)

        def drain_streams(bi):
            idx_v, vals_v, _, ssem = bufs[bi]
            for sb in range(_SSUB):
                pltpu.make_async_copy(vals_v.at[pl.ds(sb * _SIW, _SIW)],
                                      agg_sh.at[idx_v.at[sb, 0]],
                                      ssem).wait()

        def run(t, b):
            idx_v, vals_v, lsem, ssem = bufs[b]
            pltpu.make_async_copy(
                idx_hbm.at[pl.ds(irow0, _SSUB)], idx_v, lsem).wait()
            pltpu.make_async_copy(
                m_hbm.at[pl.ds(row0, _SC)], vals_v, lsem).wait()
            for sb in range(_SSUB):
                pltpu.async_copy(vals_v.at[pl.ds(sb * _SIW, _SIW)],
                                 agg_sh.at[idx_v.at[sb, 0]], ssem, add=True)

            @pl.when(t >= 1)
            def _():
                drain_streams(1 - b)

            @pl.when(t + 1 < _SNCH)
            def _():
                load(t + 1, 1 - b)

        load(0, 0)

        def chunk(u, carry):
            run(2 * u, 0)
            run(2 * u + 1, 1)
            return carry

        lax.fori_loop(0, _SNCH // 2, chunk, 0)
        run(_SNCH - 1, 0)
        drain_streams(0)

    @pl.when(c == 0)
    def _():
        do(iidx_hbm)

    @pl.when(c == 1)
    def _():
        do(jidx_hbm)

    plsc.subcore_barrier()
    pltpu.sync_copy(agg_sh.at[pl.ds(s * _NZ, _NZ)],
                    out_hbm.at[pl.ds(c * N + s * _NZ, _NZ)])

    @pl.when(s == 0)
    def _():
        pltpu.sync_copy(agg_sh.at[pl.ds(_NS * _NZ, _NZT)],
                        out_hbm.at[pl.ds(c * N + _NS * _NZ, _NZT)])


def _embed_call(node_feat, x_noisy, t2d, node_w, color_w, time_w, freqs,
                bias, wab, wba):
    return pl.pallas_call(
        _embed_body,
        out_shape=[
            jax.ShapeDtypeStruct((N, HDIM), jnp.float32),
            jax.ShapeDtypeStruct((N, 2 * HDIM), jnp.float32),
            jax.ShapeDtypeStruct((N, 2 * HDIM), jnp.float32),
        ],
    )(node_feat, x_noisy, t2d, node_w, color_w, time_w, freqs, bias, wab,
      wba)


_EB = 4000


def _edge_call(g, edge_feat, wc, b1, w2, b2):
    nblk = E // _EB
    return pl.pallas_call(
        _edge_body,
        grid=(nblk,),
        in_specs=[
            pl.BlockSpec((_EB, 2 * HDIM), lambda b: (b, 0)),
            pl.BlockSpec((_EB, EDGE_IN), lambda b: (b, 0)),
            pl.BlockSpec((EDGE_IN, HDIM), lambda b: (0, 0)),
            pl.BlockSpec((1, HDIM), lambda b: (0, 0)),
            pl.BlockSpec((HDIM, HDIM), lambda b: (0, 0)),
            pl.BlockSpec((1, HDIM), lambda b: (0, 0)),
        ],
        out_specs=pl.BlockSpec((_EB, 2 * HDIM), lambda b: (b, 0)),
        out_shape=jax.ShapeDtypeStruct((E, 2 * HDIM), jnp.float32),
    )(g, edge_feat, wc, b1, w2, b2)


def _update_call(h, agg2, w1h, w1a, b1, w2, b2, wab, wba):
    return pl.pallas_call(
        _update_body,
        out_shape=[
            jax.ShapeDtypeStruct((N, HDIM), jnp.float32),
            jax.ShapeDtypeStruct((N, 2 * HDIM), jnp.float32),
            jax.ShapeDtypeStruct((N, 2 * HDIM), jnp.float32),
        ],
    )(h, agg2, w1h, w1a, b1, w2, b2, wab, wba)


def _head_call(h, w1, b1, w2, b2):
    return pl.pallas_call(
        _head_body,
        out_shape=jax.ShapeDtypeStruct((N, 3), jnp.float32),
    )(h, w1, b1, w2, b2)


def kernel(node_feat, edge_index, edge_feat, x_noisy, t01, node_w, node_b,
           color_w, color_b, time_w, time_b, msg_w1, msg_b1, msg_w2, msg_b2,
           upd_w1, upd_b1, upd_w2, upd_b2, out_w1, out_b1, out_w2, out_b2):
    half = TIME_DIM // 2
    freqs = jnp.exp(jnp.linspace(math.log(1.0), math.log(1000.0), half))
    freqs2d = (freqs * (2.0 * math.pi)).reshape(1, half).astype(jnp.float32)
    t2d = t01.reshape(N, 1)
    bias0 = (node_b + color_b + time_b).reshape(1, HDIM)

    wa = msg_w1[:, :HDIM]
    wb = msg_w1[:, HDIM:2 * HDIM]
    wab = jnp.concatenate([wa, wb], axis=2)
    wba = jnp.concatenate([wb, wa], axis=2)

    iidx3d = edge_index[:, 0].reshape(E // _SIW, 1, _SIW)
    jidx3d = edge_index[:, 1].reshape(E // _SIW, 1, _SIW)
    zeros = jnp.zeros((_NZ, 2 * HDIM), jnp.float32)

    h, p, p2 = _embed_call(node_feat, x_noisy, t2d, node_w, color_w, time_w,
                           freqs2d, bias0, wab[0], wba[0])

    for k in range(LAYERS):
        g = _gather_call(p, p2, iidx3d, jidx3d)
        m = _edge_call(g, edge_feat, msg_w1[k, 2 * HDIM:],
                       msg_b1[k].reshape(1, HDIM), msg_w2[k],
                       msg_b2[k].reshape(1, HDIM))
        agg2 = _scatter_call(m, iidx3d, jidx3d, zeros)
        kn = min(k + 1, LAYERS - 1)
        h, p, p2 = _update_call(h, agg2, upd_w1[k, :HDIM], upd_w1[k, HDIM:],
                                upd_b1[k].reshape(1, HDIM), upd_w2[k],
                                upd_b2[k].reshape(1, HDIM), wab[kn], wba[kn])

    return _head_call(h, out_w1, out_b1.reshape(1, HDIM), out_w2,
                      out_b2.reshape(1, 3))
